# baseline (speedup 1.0000x reference)
import functools
import math

import jax
import jax.numpy as jnp
from jax.experimental import pallas as pl
from jax.experimental.pallas import tpu as pltpu

BN_EPS = 1e-5


def _affinity_kernel(x_rows_ref, xT_ref, av_ref, colsum_ref, sq_ref):
    i = pl.program_id(1)

    @pl.when(i == 0)
    def _():
        xT = xT_ref[0]
        sq_ref[...] = jnp.sum(xT * xT, axis=0, keepdims=True)

    x_blk = x_rows_ref[0]
    gram = jax.lax.dot_general(
        x_blk, xT_ref[0], (((1,), (0,)), ((), ())),
        preferred_element_type=jnp.float32)
    sq_col = jnp.sum(x_blk * x_blk, axis=1, keepdims=True)
    d2 = (sq_col + sq_ref[...]) - 2.0 * gram
    av = jnp.exp(d2 / (-2.0 * math.pi))
    av_ref[0] = av
    part = jnp.sum(av, axis=0, keepdims=True)

    @pl.when(i == 0)
    def _():
        colsum_ref[0] = part

    @pl.when(i != 0)
    def _():
        colsum_ref[0] += part


def _aggregate_kernel(av_ref, colsum_ref, xT_ref, w_ref, b_ref,
                      out_ref, ssum_ref, ssq_ref, deg_ref, xd_ref, *, bn):
    i = pl.program_id(1)

    @pl.when(i == 0)
    def _():
        deg = jax.lax.rsqrt(1.0 + colsum_ref[0])
        deg_ref[...] = deg
        xd_ref[...] = xT_ref[0] * deg

    av_blk = av_ref[0]
    m1t = jax.lax.dot_general(
        xd_ref[...], av_blk, (((1,), (1,)), ((), ())),
        preferred_element_type=jnp.float32)
    deg_blk = deg_ref[:, pl.ds(i * bn, bn)]
    xT_blk = xT_ref[0, :, pl.ds(i * bn, bn)]
    aggT = deg_blk * m1t + (deg_blk * deg_blk) * xT_blk
    avwt = jax.lax.dot_general(
        w_ref[...], aggT, (((1,), (0,)), ((), ())),
        preferred_element_type=jnp.float32) + b_ref[...]
    out_ref[0] = avwt
    psum = jnp.sum(avwt, axis=1, keepdims=True)
    psq = jnp.sum(avwt * avwt, axis=1, keepdims=True)

    @pl.when(i == 0)
    def _():
        ssum_ref[0] = psum
        ssq_ref[0] = psq

    @pl.when(i != 0)
    def _():
        ssum_ref[0] += psum
        ssq_ref[0] += psq


def _bn_kernel(avwt_ref, ssum_ref, ssq_ref, gamma_ref, beta_ref, out_ref,
               *, count):
    b_total = ssum_ref.shape[0]
    s = ssum_ref[0]
    q = ssq_ref[0]
    for bb in range(1, b_total):
        s = s + ssum_ref[bb]
        q = q + ssq_ref[bb]
    inv = 1.0 / count
    mean = s * inv
    var = q * inv - mean * mean
    scale = gamma_ref[...] * jax.lax.rsqrt(var + BN_EPS)
    shift = beta_ref[...] - mean * scale
    out_ref[0] = avwt_ref[0] * scale + shift


def kernel(x, W, b_lin, gamma, beta):
    b, c, h, w = x.shape
    n = h * w
    bn = min(256, n)
    nb = n // bn

    xT = x.reshape(b, c, n)
    x_rows = xT.transpose(0, 2, 1)

    grid1 = (b, nb)
    av, colsum = pl.pallas_call(
        _affinity_kernel,
        grid=grid1,
        in_specs=[
            pl.BlockSpec((1, bn, c), lambda bi, i: (bi, i, 0)),
            pl.BlockSpec((1, c, n), lambda bi, i: (bi, 0, 0)),
        ],
        out_specs=[
            pl.BlockSpec((1, bn, n), lambda bi, i: (bi, i, 0)),
            pl.BlockSpec((1, 1, n), lambda bi, i: (bi, 0, 0)),
        ],
        out_shape=[
            jax.ShapeDtypeStruct((b, n, n), jnp.float32),
            jax.ShapeDtypeStruct((b, 1, n), jnp.float32),
        ],
        scratch_shapes=[pltpu.VMEM((1, n), jnp.float32)],
        compiler_params=pltpu.CompilerParams(
            dimension_semantics=("parallel", "arbitrary"),
            vmem_limit_bytes=100 * 1024 * 1024,
        ),
    )(x_rows, xT)

    b2 = b_lin[:, None]
    avwt, ssum, ssq = pl.pallas_call(
        functools.partial(_aggregate_kernel, bn=bn),
        grid=(b, nb),
        in_specs=[
            pl.BlockSpec((1, bn, n), lambda bi, i: (bi, i, 0)),
            pl.BlockSpec((1, 1, n), lambda bi, i: (bi, 0, 0)),
            pl.BlockSpec((1, c, n), lambda bi, i: (bi, 0, 0)),
            pl.BlockSpec((c, c), lambda bi, i: (0, 0)),
            pl.BlockSpec((c, 1), lambda bi, i: (0, 0)),
        ],
        out_specs=[
            pl.BlockSpec((1, c, bn), lambda bi, i: (bi, 0, i)),
            pl.BlockSpec((1, c, 1), lambda bi, i: (bi, 0, 0)),
            pl.BlockSpec((1, c, 1), lambda bi, i: (bi, 0, 0)),
        ],
        out_shape=[
            jax.ShapeDtypeStruct((b, c, n), jnp.float32),
            jax.ShapeDtypeStruct((b, c, 1), jnp.float32),
            jax.ShapeDtypeStruct((b, c, 1), jnp.float32),
        ],
        scratch_shapes=[
            pltpu.VMEM((1, n), jnp.float32),
            pltpu.VMEM((c, n), jnp.float32),
        ],
        compiler_params=pltpu.CompilerParams(
            dimension_semantics=("parallel", "arbitrary"),
            vmem_limit_bytes=100 * 1024 * 1024,
        ),
    )(av, colsum, xT, W, b2)

    bn3 = min(2048, n)
    y = pl.pallas_call(
        functools.partial(_bn_kernel, count=float(b * n)),
        grid=(b, n // bn3),
        in_specs=[
            pl.BlockSpec((1, c, bn3), lambda bi, i: (bi, 0, i)),
            pl.BlockSpec((b, c, 1), lambda bi, i: (0, 0, 0)),
            pl.BlockSpec((b, c, 1), lambda bi, i: (0, 0, 0)),
            pl.BlockSpec((c, 1), lambda bi, i: (0, 0)),
            pl.BlockSpec((c, 1), lambda bi, i: (0, 0)),
        ],
        out_specs=pl.BlockSpec((1, c, bn3), lambda bi, i: (bi, 0, i)),
        out_shape=jax.ShapeDtypeStruct((b, c, n), jnp.float32),
        compiler_params=pltpu.CompilerParams(
            dimension_semantics=("parallel", "arbitrary"),
        ),
    )(avwt, ssum, ssq, gamma[:, None], beta[:, None])

    return y.reshape(b, c, h, w)


# speedup vs baseline: 1.1056x; 1.1056x over previous
import functools
import math

import jax
import jax.numpy as jnp
from jax.experimental import pallas as pl
from jax.experimental.pallas import tpu as pltpu

BN_EPS = 1e-5


def _affinity_kernel(x_rows_ref, xT_ref, av_ref, colsum_ref, sq_ref):
    i = pl.program_id(1)

    @pl.when(i == 0)
    def _():
        xT = xT_ref[0]
        sq_ref[...] = jnp.sum(xT * xT, axis=0, keepdims=True)

    x_blk = x_rows_ref[0]
    gram = jax.lax.dot_general(
        x_blk, xT_ref[0], (((1,), (0,)), ((), ())),
        preferred_element_type=jnp.float32)
    sq_col = jnp.sum(x_blk * x_blk, axis=1, keepdims=True)
    d2 = (sq_col + sq_ref[...]) - 2.0 * gram
    av = jnp.exp(d2 / (-2.0 * math.pi))
    av_ref[0] = av.astype(jnp.bfloat16)
    part = jnp.sum(av, axis=0, keepdims=True)

    @pl.when(i == 0)
    def _():
        colsum_ref[0] = part

    @pl.when(i != 0)
    def _():
        colsum_ref[0] += part


def _aggregate_kernel(av_ref, colsum_ref, xT_ref, w_ref, b_ref,
                      out_ref, ssum_ref, ssq_ref, deg_ref, xd_ref, *, bn):
    i = pl.program_id(1)

    @pl.when(i == 0)
    def _():
        deg = jax.lax.rsqrt(1.0 + colsum_ref[0])
        deg_ref[...] = deg
        xd_ref[...] = (xT_ref[0] * deg).astype(jnp.bfloat16)

    av_blk = av_ref[0]
    m1t = jax.lax.dot_general(
        xd_ref[...], av_blk, (((1,), (1,)), ((), ())),
        preferred_element_type=jnp.float32)
    deg_blk = deg_ref[:, pl.ds(i * bn, bn)]
    xT_blk = xT_ref[0, :, pl.ds(i * bn, bn)]
    aggT = deg_blk * m1t + (deg_blk * deg_blk) * xT_blk
    avwt = jax.lax.dot_general(
        w_ref[...], aggT, (((1,), (0,)), ((), ())),
        preferred_element_type=jnp.float32) + b_ref[...]
    out_ref[0] = avwt
    psum = jnp.sum(avwt, axis=1, keepdims=True)
    psq = jnp.sum(avwt * avwt, axis=1, keepdims=True)

    @pl.when(i == 0)
    def _():
        ssum_ref[0] = psum
        ssq_ref[0] = psq

    @pl.when(i != 0)
    def _():
        ssum_ref[0] += psum
        ssq_ref[0] += psq


def _bn_kernel(avwt_ref, ssum_ref, ssq_ref, gamma_ref, beta_ref, out_ref,
               *, count):
    b_total = ssum_ref.shape[0]
    s = ssum_ref[0]
    q = ssq_ref[0]
    for bb in range(1, b_total):
        s = s + ssum_ref[bb]
        q = q + ssq_ref[bb]
    inv = 1.0 / count
    mean = s * inv
    var = q * inv - mean * mean
    scale = gamma_ref[...] * jax.lax.rsqrt(var + BN_EPS)
    shift = beta_ref[...] - mean * scale
    out_ref[0] = avwt_ref[0] * scale + shift


def kernel(x, W, b_lin, gamma, beta):
    b, c, h, w = x.shape
    n = h * w
    bn = min(256, n)
    nb = n // bn

    xT = x.reshape(b, c, n)
    x_rows = xT.transpose(0, 2, 1)

    grid1 = (b, nb)
    av, colsum = pl.pallas_call(
        _affinity_kernel,
        grid=grid1,
        in_specs=[
            pl.BlockSpec((1, bn, c), lambda bi, i: (bi, i, 0)),
            pl.BlockSpec((1, c, n), lambda bi, i: (bi, 0, 0)),
        ],
        out_specs=[
            pl.BlockSpec((1, bn, n), lambda bi, i: (bi, i, 0)),
            pl.BlockSpec((1, 1, n), lambda bi, i: (bi, 0, 0)),
        ],
        out_shape=[
            jax.ShapeDtypeStruct((b, n, n), jnp.bfloat16),
            jax.ShapeDtypeStruct((b, 1, n), jnp.float32),
        ],
        scratch_shapes=[pltpu.VMEM((1, n), jnp.float32)],
        compiler_params=pltpu.CompilerParams(
            dimension_semantics=("parallel", "arbitrary"),
            vmem_limit_bytes=100 * 1024 * 1024,
        ),
    )(x_rows, xT)

    b2 = b_lin[:, None]
    avwt, ssum, ssq = pl.pallas_call(
        functools.partial(_aggregate_kernel, bn=bn),
        grid=(b, nb),
        in_specs=[
            pl.BlockSpec((1, bn, n), lambda bi, i: (bi, i, 0)),
            pl.BlockSpec((1, 1, n), lambda bi, i: (bi, 0, 0)),
            pl.BlockSpec((1, c, n), lambda bi, i: (bi, 0, 0)),
            pl.BlockSpec((c, c), lambda bi, i: (0, 0)),
            pl.BlockSpec((c, 1), lambda bi, i: (0, 0)),
        ],
        out_specs=[
            pl.BlockSpec((1, c, bn), lambda bi, i: (bi, 0, i)),
            pl.BlockSpec((1, c, 1), lambda bi, i: (bi, 0, 0)),
            pl.BlockSpec((1, c, 1), lambda bi, i: (bi, 0, 0)),
        ],
        out_shape=[
            jax.ShapeDtypeStruct((b, c, n), jnp.float32),
            jax.ShapeDtypeStruct((b, c, 1), jnp.float32),
            jax.ShapeDtypeStruct((b, c, 1), jnp.float32),
        ],
        scratch_shapes=[
            pltpu.VMEM((1, n), jnp.float32),
            pltpu.VMEM((c, n), jnp.bfloat16),
        ],
        compiler_params=pltpu.CompilerParams(
            dimension_semantics=("parallel", "arbitrary"),
            vmem_limit_bytes=100 * 1024 * 1024,
        ),
    )(av, colsum, xT, W, b2)

    bn3 = min(2048, n)
    y = pl.pallas_call(
        functools.partial(_bn_kernel, count=float(b * n)),
        grid=(b, n // bn3),
        in_specs=[
            pl.BlockSpec((1, c, bn3), lambda bi, i: (bi, 0, i)),
            pl.BlockSpec((b, c, 1), lambda bi, i: (0, 0, 0)),
            pl.BlockSpec((b, c, 1), lambda bi, i: (0, 0, 0)),
            pl.BlockSpec((c, 1), lambda bi, i: (0, 0)),
            pl.BlockSpec((c, 1), lambda bi, i: (0, 0)),
        ],
        out_specs=pl.BlockSpec((1, c, bn3), lambda bi, i: (bi, 0, i)),
        out_shape=jax.ShapeDtypeStruct((b, c, n), jnp.float32),
        compiler_params=pltpu.CompilerParams(
            dimension_semantics=("parallel", "arbitrary"),
        ),
    )(avwt, ssum, ssq, gamma[:, None], beta[:, None])

    return y.reshape(b, c, h, w)


# speedup vs baseline: 1.1584x; 1.0478x over previous
import functools
import math

import jax
import jax.numpy as jnp
from jax.experimental import pallas as pl
from jax.experimental.pallas import tpu as pltpu

BN_EPS = 1e-5


def _affinity_kernel(x_rows_ref, xTb_ref, xT_ref, av_ref, colsum_ref,
                     hsq_ref):
    i = pl.program_id(1)

    @pl.when(i == 0)
    def _():
        xT = xT_ref[0]
        hsq_ref[...] = 0.5 * jnp.sum(xT * xT, axis=0, keepdims=True)

    x_blk = x_rows_ref[0]
    gram = jax.lax.dot_general(
        x_blk.astype(jnp.bfloat16), xTb_ref[0], (((1,), (0,)), ((), ())),
        preferred_element_type=jnp.float32)
    hsq_col = 0.5 * jnp.sum(x_blk * x_blk, axis=1, keepdims=True)
    z = (gram - hsq_col) - hsq_ref[...]
    av = jnp.exp2(z * (1.4426950408889634 / math.pi))
    av_ref[0] = av.astype(jnp.bfloat16)
    part = jnp.sum(av, axis=0, keepdims=True)

    @pl.when(i == 0)
    def _():
        colsum_ref[0] = part

    @pl.when(i != 0)
    def _():
        colsum_ref[0] += part


def _aggregate_kernel(av_ref, colsum_ref, xT_ref, w_ref, b_ref,
                      out_ref, ssum_ref, ssq_ref, deg_ref, xd_ref, *, bn):
    i = pl.program_id(1)

    @pl.when(i == 0)
    def _():
        deg = jax.lax.rsqrt(1.0 + colsum_ref[0])
        deg_ref[...] = deg
        xd_ref[...] = (xT_ref[0] * deg).astype(jnp.bfloat16)

    av_col = av_ref[0]
    m1t = jax.lax.dot_general(
        xd_ref[...], av_col, (((1,), (0,)), ((), ())),
        preferred_element_type=jnp.float32)
    deg_blk = deg_ref[:, pl.ds(i * bn, bn)]
    xT_blk = xT_ref[0, :, pl.ds(i * bn, bn)]
    aggT = deg_blk * m1t + (deg_blk * deg_blk) * xT_blk
    avwt = jax.lax.dot_general(
        w_ref[...], aggT, (((1,), (0,)), ((), ())),
        preferred_element_type=jnp.float32) + b_ref[...]
    out_ref[0] = avwt
    psum = jnp.sum(avwt, axis=1, keepdims=True)
    psq = jnp.sum(avwt * avwt, axis=1, keepdims=True)

    @pl.when(i == 0)
    def _():
        ssum_ref[0] = psum
        ssq_ref[0] = psq

    @pl.when(i != 0)
    def _():
        ssum_ref[0] += psum
        ssq_ref[0] += psq


def _bn_kernel(avwt_ref, ssum_ref, ssq_ref, gamma_ref, beta_ref, out_ref,
               *, count):
    b_total = ssum_ref.shape[0]
    s = ssum_ref[0]
    q = ssq_ref[0]
    for bb in range(1, b_total):
        s = s + ssum_ref[bb]
        q = q + ssq_ref[bb]
    inv = 1.0 / count
    mean = s * inv
    var = q * inv - mean * mean
    scale = gamma_ref[...] * jax.lax.rsqrt(var + BN_EPS)
    shift = beta_ref[...] - mean * scale
    out_ref[0] = avwt_ref[0] * scale + shift


def kernel(x, W, b_lin, gamma, beta):
    b, c, h, w = x.shape
    n = h * w
    bn = min(256, n)
    nb = n // bn

    xT = x.reshape(b, c, n)
    x_rows = xT.transpose(0, 2, 1)

    grid1 = (b, nb)
    av, colsum = pl.pallas_call(
        _affinity_kernel,
        grid=grid1,
        in_specs=[
            pl.BlockSpec((1, bn, c), lambda bi, i: (bi, i, 0)),
            pl.BlockSpec((1, c, n), lambda bi, i: (bi, 0, 0)),
            pl.BlockSpec((1, c, n), lambda bi, i: (bi, 0, 0)),
        ],
        out_specs=[
            pl.BlockSpec((1, bn, n), lambda bi, i: (bi, i, 0)),
            pl.BlockSpec((1, 1, n), lambda bi, i: (bi, 0, 0)),
        ],
        out_shape=[
            jax.ShapeDtypeStruct((b, n, n), jnp.bfloat16),
            jax.ShapeDtypeStruct((b, 1, n), jnp.float32),
        ],
        scratch_shapes=[pltpu.VMEM((1, n), jnp.float32)],
        compiler_params=pltpu.CompilerParams(
            dimension_semantics=("arbitrary", "arbitrary"),
            vmem_limit_bytes=100 * 1024 * 1024,
        ),
    )(x_rows, xT.astype(jnp.bfloat16), xT)

    b2 = b_lin[:, None]
    avwt, ssum, ssq = pl.pallas_call(
        functools.partial(_aggregate_kernel, bn=bn),
        grid=(b, nb),
        in_specs=[
            pl.BlockSpec((1, n, bn), lambda bi, i: (bi, 0, i)),
            pl.BlockSpec((1, 1, n), lambda bi, i: (bi, 0, 0)),
            pl.BlockSpec((1, c, n), lambda bi, i: (bi, 0, 0)),
            pl.BlockSpec((c, c), lambda bi, i: (0, 0)),
            pl.BlockSpec((c, 1), lambda bi, i: (0, 0)),
        ],
        out_specs=[
            pl.BlockSpec((1, c, bn), lambda bi, i: (bi, 0, i)),
            pl.BlockSpec((1, c, 1), lambda bi, i: (bi, 0, 0)),
            pl.BlockSpec((1, c, 1), lambda bi, i: (bi, 0, 0)),
        ],
        out_shape=[
            jax.ShapeDtypeStruct((b, c, n), jnp.float32),
            jax.ShapeDtypeStruct((b, c, 1), jnp.float32),
            jax.ShapeDtypeStruct((b, c, 1), jnp.float32),
        ],
        scratch_shapes=[
            pltpu.VMEM((1, n), jnp.float32),
            pltpu.VMEM((c, n), jnp.bfloat16),
        ],
        compiler_params=pltpu.CompilerParams(
            dimension_semantics=("arbitrary", "arbitrary"),
            vmem_limit_bytes=100 * 1024 * 1024,
        ),
    )(av, colsum, xT, W, b2)

    bn3 = min(2048, n)
    y = pl.pallas_call(
        functools.partial(_bn_kernel, count=float(b * n)),
        grid=(b, n // bn3),
        in_specs=[
            pl.BlockSpec((1, c, bn3), lambda bi, i: (bi, 0, i)),
            pl.BlockSpec((b, c, 1), lambda bi, i: (0, 0, 0)),
            pl.BlockSpec((b, c, 1), lambda bi, i: (0, 0, 0)),
            pl.BlockSpec((c, 1), lambda bi, i: (0, 0)),
            pl.BlockSpec((c, 1), lambda bi, i: (0, 0)),
        ],
        out_specs=pl.BlockSpec((1, c, bn3), lambda bi, i: (bi, 0, i)),
        out_shape=jax.ShapeDtypeStruct((b, c, n), jnp.float32),
        compiler_params=pltpu.CompilerParams(
            dimension_semantics=("arbitrary", "arbitrary"),
        ),
    )(avwt, ssum, ssq, gamma[:, None], beta[:, None])

    return y.reshape(b, c, h, w)


# speedup vs baseline: 1.3400x; 1.1567x over previous
import functools
import math

import jax
import jax.numpy as jnp
from jax.experimental import pallas as pl
from jax.experimental.pallas import tpu as pltpu

BN_EPS = 1e-5
_C2 = 1.4426950408889634 / math.pi


def _main_kernel(xT_ref, w_ref, b_ref,
                 out_ref, ssum_ref, ssq_ref,
                 av_ref, hsq_ref, colsum_ref, deg_ref, xb_ref, xd_ref,
                 *, bn, nb):
    p = pl.program_id(1)
    i = pl.program_id(2)

    @pl.when((p == 0) & (i == 0))
    def _():
        xT = xT_ref[0]
        hsq_ref[...] = 0.5 * jnp.sum(xT * xT, axis=0, keepdims=True)
        xb_ref[...] = xT.astype(jnp.bfloat16)

    @pl.when(p == 0)
    def _():
        xb_blk = xb_ref[:, pl.ds(i * bn, bn)]
        gram = jax.lax.dot_general(
            xb_blk, xb_ref[...], (((0,), (0,)), ((), ())),
            preferred_element_type=jnp.float32)
        hsq_col = hsq_ref[:, pl.ds(i * bn, bn)].T
        z = (gram - hsq_col) - hsq_ref[...]
        av = jnp.exp2(z * _C2)
        av_ref[pl.ds(i * bn, bn), :] = av.astype(jnp.bfloat16)
        part = jnp.sum(av, axis=0, keepdims=True)

        @pl.when(i == 0)
        def _():
            colsum_ref[...] = part

        @pl.when(i != 0)
        def _():
            colsum_ref[...] += part

    @pl.when(p == 1)
    def _():
        @pl.when(i == 0)
        def _():
            deg = jax.lax.rsqrt(1.0 + colsum_ref[...])
            deg_ref[...] = deg
            xd_ref[...] = (xT_ref[0] * deg).astype(jnp.bfloat16)

        av_col = av_ref[:, pl.ds(i * bn, bn)]
        m1t = jax.lax.dot_general(
            xd_ref[...], av_col, (((1,), (0,)), ((), ())),
            preferred_element_type=jnp.float32)
        deg_blk = deg_ref[:, pl.ds(i * bn, bn)]
        xT_blk = xT_ref[0, :, pl.ds(i * bn, bn)]
        aggT = deg_blk * m1t + (deg_blk * deg_blk) * xT_blk
        avwt = jax.lax.dot_general(
            w_ref[...], aggT, (((1,), (0,)), ((), ())),
            preferred_element_type=jnp.float32) + b_ref[...]
        out_ref[0] = avwt
        psum = jnp.sum(avwt, axis=1, keepdims=True)
        psq = jnp.sum(avwt * avwt, axis=1, keepdims=True)

        @pl.when(i == 0)
        def _():
            ssum_ref[0] = psum
            ssq_ref[0] = psq

        @pl.when(i != 0)
        def _():
            ssum_ref[0] += psum
            ssq_ref[0] += psq


def _bn_kernel(avwt_ref, ssum_ref, ssq_ref, gamma_ref, beta_ref, out_ref,
               *, count):
    b_total = ssum_ref.shape[0]
    s = ssum_ref[0]
    q = ssq_ref[0]
    for bb in range(1, b_total):
        s = s + ssum_ref[bb]
        q = q + ssq_ref[bb]
    inv = 1.0 / count
    mean = s * inv
    var = q * inv - mean * mean
    scale = gamma_ref[...] * jax.lax.rsqrt(var + BN_EPS)
    shift = beta_ref[...] - mean * scale
    out_ref[0] = avwt_ref[0] * scale + shift


def kernel(x, W, b_lin, gamma, beta):
    b, c, h, w = x.shape
    n = h * w
    bn = min(256, n)
    nb = n // bn

    xT = x.reshape(b, c, n)

    avwt, ssum, ssq = pl.pallas_call(
        functools.partial(_main_kernel, bn=bn, nb=nb),
        grid=(b, 2, nb),
        in_specs=[
            pl.BlockSpec((1, c, n), lambda bi, p, i: (bi, 0, 0)),
            pl.BlockSpec((c, c), lambda bi, p, i: (0, 0)),
            pl.BlockSpec((c, 1), lambda bi, p, i: (0, 0)),
        ],
        out_specs=[
            pl.BlockSpec((1, c, bn), lambda bi, p, i: (bi, 0, i * p)),
            pl.BlockSpec((1, c, 1), lambda bi, p, i: (bi, 0, 0)),
            pl.BlockSpec((1, c, 1), lambda bi, p, i: (bi, 0, 0)),
        ],
        out_shape=[
            jax.ShapeDtypeStruct((b, c, n), jnp.float32),
            jax.ShapeDtypeStruct((b, c, 1), jnp.float32),
            jax.ShapeDtypeStruct((b, c, 1), jnp.float32),
        ],
        scratch_shapes=[
            pltpu.VMEM((n, n), jnp.bfloat16),
            pltpu.VMEM((1, n), jnp.float32),
            pltpu.VMEM((1, n), jnp.float32),
            pltpu.VMEM((1, n), jnp.float32),
            pltpu.VMEM((c, n), jnp.bfloat16),
            pltpu.VMEM((c, n), jnp.bfloat16),
        ],
        compiler_params=pltpu.CompilerParams(
            dimension_semantics=("arbitrary", "arbitrary", "arbitrary"),
            vmem_limit_bytes=100 * 1024 * 1024,
        ),
    )(xT, W, b_lin[:, None])

    bn3 = min(2048, n)
    y = pl.pallas_call(
        functools.partial(_bn_kernel, count=float(b * n)),
        grid=(b, n // bn3),
        in_specs=[
            pl.BlockSpec((1, c, bn3), lambda bi, i: (bi, 0, i)),
            pl.BlockSpec((b, c, 1), lambda bi, i: (0, 0, 0)),
            pl.BlockSpec((b, c, 1), lambda bi, i: (0, 0, 0)),
            pl.BlockSpec((c, 1), lambda bi, i: (0, 0)),
            pl.BlockSpec((c, 1), lambda bi, i: (0, 0)),
        ],
        out_specs=pl.BlockSpec((1, c, bn3), lambda bi, i: (bi, 0, i)),
        out_shape=jax.ShapeDtypeStruct((b, c, n), jnp.float32),
        compiler_params=pltpu.CompilerParams(
            dimension_semantics=("arbitrary", "arbitrary"),
        ),
    )(avwt, ssum, ssq, gamma[:, None], beta[:, None])

    return y.reshape(b, c, h, w)


# speedup vs baseline: 1.5520x; 1.1582x over previous
import functools
import math

import jax
import jax.numpy as jnp
from jax.experimental import pallas as pl
from jax.experimental.pallas import tpu as pltpu

BN_EPS = 1e-5
_C2 = 1.4426950408889634 / math.pi


def _main_kernel(xT_ref, w_ref, b_ref,
                 out_ref, ssum_ref, ssq_ref,
                 av_ref, hsq_ref, colsum_ref, deg_ref, xb_ref, xd_ref,
                 *, bn, nb):
    p = pl.program_id(1)
    i = pl.program_id(2)

    @pl.when((p == 0) & (i == 0))
    def _():
        xT = xT_ref[0]
        hsq_ref[...] = 0.5 * jnp.sum(xT * xT, axis=0, keepdims=True)
        xb_ref[...] = xT.astype(jnp.bfloat16)

    @pl.when(p == 0)
    def _():
        xb_blk = xb_ref[:, pl.ds(i * bn, bn)]
        gram = jax.lax.dot_general(
            xb_blk, xb_ref[...], (((0,), (0,)), ((), ())),
            preferred_element_type=jnp.float32)
        hsq_col = hsq_ref[:, pl.ds(i * bn, bn)].T
        z = (gram - hsq_col) - hsq_ref[...]
        av = jnp.exp2(z * _C2)
        av_ref[pl.ds(i * bn, bn), :] = av.astype(jnp.bfloat16)
        part = jnp.sum(av, axis=0, keepdims=True)

        @pl.when(i == 0)
        def _():
            colsum_ref[...] = part

        @pl.when(i != 0)
        def _():
            colsum_ref[...] += part

    @pl.when(p == 1)
    def _():
        @pl.when(i == 0)
        def _():
            deg = jax.lax.rsqrt(1.0 + colsum_ref[...])
            deg_ref[...] = deg
            xd_ref[...] = (xT_ref[0] * deg).astype(jnp.bfloat16)

        av_col = av_ref[:, pl.ds(i * bn, bn)]
        m1t = jax.lax.dot_general(
            xd_ref[...], av_col, (((1,), (0,)), ((), ())),
            preferred_element_type=jnp.float32)
        deg_blk = deg_ref[:, pl.ds(i * bn, bn)]
        xT_blk = xT_ref[0, :, pl.ds(i * bn, bn)]
        aggT = deg_blk * m1t + (deg_blk * deg_blk) * xT_blk
        avwt = jax.lax.dot_general(
            w_ref[...], aggT, (((1,), (0,)), ((), ())),
            preferred_element_type=jnp.float32) + b_ref[...]
        out_ref[0] = avwt
        psum = jnp.sum(avwt, axis=1, keepdims=True)
        psq = jnp.sum(avwt * avwt, axis=1, keepdims=True)

        @pl.when(i == 0)
        def _():
            ssum_ref[0] = psum
            ssq_ref[0] = psq

        @pl.when(i != 0)
        def _():
            ssum_ref[0] += psum
            ssq_ref[0] += psq


def _bn_kernel(avwt_ref, ssum_ref, ssq_ref, gamma_ref, beta_ref, out_ref,
               *, count):
    b_total = ssum_ref.shape[0]
    s = ssum_ref[0]
    q = ssq_ref[0]
    for bb in range(1, b_total):
        s = s + ssum_ref[bb]
        q = q + ssq_ref[bb]
    inv = 1.0 / count
    mean = s * inv
    var = q * inv - mean * mean
    scale = gamma_ref[...] * jax.lax.rsqrt(var + BN_EPS)
    shift = beta_ref[...] - mean * scale
    out_ref[0] = avwt_ref[0] * scale + shift


def kernel(x, W, b_lin, gamma, beta):
    b, c, h, w = x.shape
    n = h * w
    bn = min(512, n)
    nb = n // bn

    xT = x.reshape(b, c, n)

    avwt, ssum, ssq = pl.pallas_call(
        functools.partial(_main_kernel, bn=bn, nb=nb),
        grid=(b, 2, nb),
        in_specs=[
            pl.BlockSpec((1, c, n), lambda bi, p, i: (bi, 0, 0)),
            pl.BlockSpec((c, c), lambda bi, p, i: (0, 0)),
            pl.BlockSpec((c, 1), lambda bi, p, i: (0, 0)),
        ],
        out_specs=[
            pl.BlockSpec((1, c, bn), lambda bi, p, i: (bi, 0, i * p)),
            pl.BlockSpec((1, c, 1), lambda bi, p, i: (bi, 0, 0)),
            pl.BlockSpec((1, c, 1), lambda bi, p, i: (bi, 0, 0)),
        ],
        out_shape=[
            jax.ShapeDtypeStruct((b, c, n), jnp.float32),
            jax.ShapeDtypeStruct((b, c, 1), jnp.float32),
            jax.ShapeDtypeStruct((b, c, 1), jnp.float32),
        ],
        scratch_shapes=[
            pltpu.VMEM((n, n), jnp.bfloat16),
            pltpu.VMEM((1, n), jnp.float32),
            pltpu.VMEM((1, n), jnp.float32),
            pltpu.VMEM((1, n), jnp.float32),
            pltpu.VMEM((c, n), jnp.bfloat16),
            pltpu.VMEM((c, n), jnp.bfloat16),
        ],
        compiler_params=pltpu.CompilerParams(
            dimension_semantics=("arbitrary", "arbitrary", "arbitrary"),
            vmem_limit_bytes=100 * 1024 * 1024,
        ),
    )(xT, W, b_lin[:, None])

    bn3 = min(4096, n)
    y = pl.pallas_call(
        functools.partial(_bn_kernel, count=float(b * n)),
        grid=(b, n // bn3),
        in_specs=[
            pl.BlockSpec((1, c, bn3), lambda bi, i: (bi, 0, i)),
            pl.BlockSpec((b, c, 1), lambda bi, i: (0, 0, 0)),
            pl.BlockSpec((b, c, 1), lambda bi, i: (0, 0, 0)),
            pl.BlockSpec((c, 1), lambda bi, i: (0, 0)),
            pl.BlockSpec((c, 1), lambda bi, i: (0, 0)),
        ],
        out_specs=pl.BlockSpec((1, c, bn3), lambda bi, i: (bi, 0, i)),
        out_shape=jax.ShapeDtypeStruct((b, c, n), jnp.float32),
        compiler_params=pltpu.CompilerParams(
            dimension_semantics=("arbitrary", "arbitrary"),
        ),
    )(avwt, ssum, ssq, gamma[:, None], beta[:, None])

    return y.reshape(b, c, h, w)


# speedup vs baseline: 1.7452x; 1.1245x over previous
import functools
import math

import jax
import jax.numpy as jnp
from jax.experimental import pallas as pl
from jax.experimental.pallas import tpu as pltpu

BN_EPS = 1e-5
_C2 = 1.4426950408889634 / math.pi


def _main_kernel(xT_ref, w_ref, b_ref,
                 out_ref, ssum_ref, ssq_ref,
                 av_ref, hsq_ref, colsum_ref, deg_ref, xb_ref, xd_ref,
                 *, bn, nb):
    p = pl.program_id(1)
    i = pl.program_id(2)

    @pl.when((p == 0) & (i == 0))
    def _():
        xT = xT_ref[0]
        hsq_ref[...] = 0.5 * jnp.sum(xT * xT, axis=0, keepdims=True)
        xb_ref[...] = xT.astype(jnp.bfloat16)

    @pl.when(p == 0)
    def _():
        xb_blk = xb_ref[:, pl.ds(i * bn, bn)]
        gram = jax.lax.dot_general(
            xb_blk, xb_ref[...], (((0,), (0,)), ((), ())),
            preferred_element_type=jnp.float32)
        hsq_col = hsq_ref[:, pl.ds(i * bn, bn)].T
        z = (gram - hsq_col) - hsq_ref[...]
        av = jnp.exp2(z * _C2)
        av_ref[pl.ds(i * bn, bn), :] = av.astype(jnp.bfloat16)
        part = jnp.sum(av, axis=0, keepdims=True)

        @pl.when(i == 0)
        def _():
            colsum_ref[...] = part

        @pl.when(i != 0)
        def _():
            colsum_ref[...] += part

    @pl.when(p == 1)
    def _():
        @pl.when(i == 0)
        def _():
            deg = jax.lax.rsqrt(1.0 + colsum_ref[...])
            deg_ref[...] = deg
            xd_ref[...] = (xT_ref[0] * deg).astype(jnp.bfloat16)

        av_col = av_ref[:, pl.ds(i * bn, bn)]
        m1t = jax.lax.dot_general(
            xd_ref[...], av_col, (((1,), (0,)), ((), ())),
            preferred_element_type=jnp.float32)
        deg_blk = deg_ref[:, pl.ds(i * bn, bn)]
        xT_blk = xT_ref[0, :, pl.ds(i * bn, bn)]
        aggT = deg_blk * m1t + (deg_blk * deg_blk) * xT_blk
        avwt = jax.lax.dot_general(
            w_ref[...], aggT, (((1,), (0,)), ((), ())),
            preferred_element_type=jnp.float32) + b_ref[...]
        out_ref[0] = avwt
        psum = jnp.sum(avwt, axis=1, keepdims=True)
        psq = jnp.sum(avwt * avwt, axis=1, keepdims=True)

        @pl.when(i == 0)
        def _():
            ssum_ref[0] = psum
            ssq_ref[0] = psq

        @pl.when(i != 0)
        def _():
            ssum_ref[0] += psum
            ssq_ref[0] += psq


def _bn_kernel(avwt_ref, ssum_ref, ssq_ref, gamma_ref, beta_ref, out_ref,
               *, count):
    b_total = ssum_ref.shape[0]
    s = ssum_ref[0]
    q = ssq_ref[0]
    for bb in range(1, b_total):
        s = s + ssum_ref[bb]
        q = q + ssq_ref[bb]
    inv = 1.0 / count
    mean = s * inv
    var = q * inv - mean * mean
    scale = gamma_ref[...] * jax.lax.rsqrt(var + BN_EPS)
    shift = beta_ref[...] - mean * scale
    out_ref[0] = avwt_ref[0] * scale + shift


def kernel(x, W, b_lin, gamma, beta):
    b, c, h, w = x.shape
    n = h * w
    bn = min(1024, n)
    nb = n // bn

    xT = x.reshape(b, c, n)

    avwt, ssum, ssq = pl.pallas_call(
        functools.partial(_main_kernel, bn=bn, nb=nb),
        grid=(b, 2, nb),
        in_specs=[
            pl.BlockSpec((1, c, n), lambda bi, p, i: (bi, 0, 0)),
            pl.BlockSpec((c, c), lambda bi, p, i: (0, 0)),
            pl.BlockSpec((c, 1), lambda bi, p, i: (0, 0)),
        ],
        out_specs=[
            pl.BlockSpec((1, c, bn), lambda bi, p, i: (bi, 0, i * p)),
            pl.BlockSpec((1, c, 1), lambda bi, p, i: (bi, 0, 0)),
            pl.BlockSpec((1, c, 1), lambda bi, p, i: (bi, 0, 0)),
        ],
        out_shape=[
            jax.ShapeDtypeStruct((b, c, n), jnp.float32),
            jax.ShapeDtypeStruct((b, c, 1), jnp.float32),
            jax.ShapeDtypeStruct((b, c, 1), jnp.float32),
        ],
        scratch_shapes=[
            pltpu.VMEM((n, n), jnp.bfloat16),
            pltpu.VMEM((1, n), jnp.float32),
            pltpu.VMEM((1, n), jnp.float32),
            pltpu.VMEM((1, n), jnp.float32),
            pltpu.VMEM((c, n), jnp.bfloat16),
            pltpu.VMEM((c, n), jnp.bfloat16),
        ],
        compiler_params=pltpu.CompilerParams(
            dimension_semantics=("arbitrary", "arbitrary", "arbitrary"),
            vmem_limit_bytes=100 * 1024 * 1024,
        ),
    )(xT, W, b_lin[:, None])

    bn3 = min(4096, n)
    y = pl.pallas_call(
        functools.partial(_bn_kernel, count=float(b * n)),
        grid=(b, n // bn3),
        in_specs=[
            pl.BlockSpec((1, c, bn3), lambda bi, i: (bi, 0, i)),
            pl.BlockSpec((b, c, 1), lambda bi, i: (0, 0, 0)),
            pl.BlockSpec((b, c, 1), lambda bi, i: (0, 0, 0)),
            pl.BlockSpec((c, 1), lambda bi, i: (0, 0)),
            pl.BlockSpec((c, 1), lambda bi, i: (0, 0)),
        ],
        out_specs=pl.BlockSpec((1, c, bn3), lambda bi, i: (bi, 0, i)),
        out_shape=jax.ShapeDtypeStruct((b, c, n), jnp.float32),
        compiler_params=pltpu.CompilerParams(
            dimension_semantics=("arbitrary", "arbitrary"),
        ),
    )(avwt, ssum, ssq, gamma[:, None], beta[:, None])

    return y.reshape(b, c, h, w)


# speedup vs baseline: 1.8471x; 1.0584x over previous
import functools
import math

import jax
import jax.numpy as jnp
from jax.experimental import pallas as pl
from jax.experimental.pallas import tpu as pltpu

BN_EPS = 1e-5
_C2 = 1.4426950408889634 / math.pi


def _main_kernel(xT_ref, w_ref, b_ref,
                 out_ref, ssum_ref, ssq_ref,
                 av_ref, hsq_ref, colsum_ref, deg_ref, xb_ref, xd_ref,
                 *, bn, nb):
    p = pl.program_id(1)
    i = pl.program_id(2)

    @pl.when((p == 0) & (i == 0))
    def _():
        xT = xT_ref[0]
        hsq_ref[...] = 0.5 * jnp.sum(xT * xT, axis=0, keepdims=True)
        xb_ref[...] = xT.astype(jnp.bfloat16)

    @pl.when(p == 0)
    def _():
        xb_blk = xb_ref[:, pl.ds(i * bn, bn)]
        gram = jax.lax.dot_general(
            xb_blk, xb_ref[...], (((0,), (0,)), ((), ())),
            preferred_element_type=jnp.float32)
        hsq_col = hsq_ref[:, pl.ds(i * bn, bn)].T
        z = (gram - hsq_col) - hsq_ref[...]
        av = jnp.exp2(z * _C2)
        av_ref[pl.ds(i * bn, bn), :] = av.astype(jnp.bfloat16)
        part = jnp.sum(av, axis=0, keepdims=True)

        @pl.when(i == 0)
        def _():
            colsum_ref[...] = part

        @pl.when(i != 0)
        def _():
            colsum_ref[...] += part

    @pl.when(p == 1)
    def _():
        @pl.when(i == 0)
        def _():
            deg = jax.lax.rsqrt(1.0 + colsum_ref[...])
            deg_ref[...] = deg
            xd_ref[...] = (xT_ref[0] * deg).astype(jnp.bfloat16)

        av_col = av_ref[:, pl.ds(i * bn, bn)]
        m1t = jax.lax.dot_general(
            xd_ref[...], av_col, (((1,), (0,)), ((), ())),
            preferred_element_type=jnp.float32)
        deg_blk = deg_ref[:, pl.ds(i * bn, bn)]
        xT_blk = xT_ref[0, :, pl.ds(i * bn, bn)]
        aggT = deg_blk * m1t + (deg_blk * deg_blk) * xT_blk
        avwt = jax.lax.dot_general(
            w_ref[...], aggT, (((1,), (0,)), ((), ())),
            preferred_element_type=jnp.float32) + b_ref[...]
        out_ref[0] = avwt
        psum = jnp.sum(avwt, axis=1, keepdims=True)
        psq = jnp.sum(avwt * avwt, axis=1, keepdims=True)

        @pl.when(i == 0)
        def _():
            ssum_ref[0] = psum
            ssq_ref[0] = psq

        @pl.when(i != 0)
        def _():
            ssum_ref[0] += psum
            ssq_ref[0] += psq


def _bn_kernel(avwt_ref, ssum_ref, ssq_ref, gamma_ref, beta_ref, out_ref,
               *, count):
    b_total = ssum_ref.shape[0]
    s = ssum_ref[0]
    q = ssq_ref[0]
    for bb in range(1, b_total):
        s = s + ssum_ref[bb]
        q = q + ssq_ref[bb]
    inv = 1.0 / count
    mean = s * inv
    var = q * inv - mean * mean
    scale = gamma_ref[...] * jax.lax.rsqrt(var + BN_EPS)
    shift = beta_ref[...] - mean * scale
    out_ref[0] = avwt_ref[0] * scale + shift


def kernel(x, W, b_lin, gamma, beta):
    b, c, h, w = x.shape
    n = h * w
    bn = min(2048, n)
    nb = n // bn

    xT = x.reshape(b, c, n)

    avwt, ssum, ssq = pl.pallas_call(
        functools.partial(_main_kernel, bn=bn, nb=nb),
        grid=(b, 2, nb),
        in_specs=[
            pl.BlockSpec((1, c, n), lambda bi, p, i: (bi, 0, 0)),
            pl.BlockSpec((c, c), lambda bi, p, i: (0, 0)),
            pl.BlockSpec((c, 1), lambda bi, p, i: (0, 0)),
        ],
        out_specs=[
            pl.BlockSpec((1, c, bn), lambda bi, p, i: (bi, 0, i * p)),
            pl.BlockSpec((1, c, 1), lambda bi, p, i: (bi, 0, 0)),
            pl.BlockSpec((1, c, 1), lambda bi, p, i: (bi, 0, 0)),
        ],
        out_shape=[
            jax.ShapeDtypeStruct((b, c, n), jnp.float32),
            jax.ShapeDtypeStruct((b, c, 1), jnp.float32),
            jax.ShapeDtypeStruct((b, c, 1), jnp.float32),
        ],
        scratch_shapes=[
            pltpu.VMEM((n, n), jnp.bfloat16),
            pltpu.VMEM((1, n), jnp.float32),
            pltpu.VMEM((1, n), jnp.float32),
            pltpu.VMEM((1, n), jnp.float32),
            pltpu.VMEM((c, n), jnp.bfloat16),
            pltpu.VMEM((c, n), jnp.bfloat16),
        ],
        compiler_params=pltpu.CompilerParams(
            dimension_semantics=("arbitrary", "arbitrary", "arbitrary"),
            vmem_limit_bytes=100 * 1024 * 1024,
        ),
    )(xT, W, b_lin[:, None])

    bn3 = min(4096, n)
    y = pl.pallas_call(
        functools.partial(_bn_kernel, count=float(b * n)),
        grid=(b, n // bn3),
        in_specs=[
            pl.BlockSpec((1, c, bn3), lambda bi, i: (bi, 0, i)),
            pl.BlockSpec((b, c, 1), lambda bi, i: (0, 0, 0)),
            pl.BlockSpec((b, c, 1), lambda bi, i: (0, 0, 0)),
            pl.BlockSpec((c, 1), lambda bi, i: (0, 0)),
            pl.BlockSpec((c, 1), lambda bi, i: (0, 0)),
        ],
        out_specs=pl.BlockSpec((1, c, bn3), lambda bi, i: (bi, 0, i)),
        out_shape=jax.ShapeDtypeStruct((b, c, n), jnp.float32),
        compiler_params=pltpu.CompilerParams(
            dimension_semantics=("arbitrary", "arbitrary"),
        ),
    )(avwt, ssum, ssq, gamma[:, None], beta[:, None])

    return y.reshape(b, c, h, w)
